# baseline (speedup 1.0000x reference)
import functools

import jax
import jax.numpy as jnp
from jax.experimental import pallas as pl
from jax.experimental.pallas import tpu as pltpu

N = 4096
F = 128
TM = 512
TK = 512
TN = 512


def _prep_body(nif_ref, g_ref, b_ref, mw_ref, mb_ref, w_ref, sq_ref):
    z = nif_ref[...]
    m = jnp.mean(z, axis=0, keepdims=True)
    v = jnp.mean((z - m) * (z - m), axis=0, keepdims=True)
    zn = (z - m) / jnp.sqrt(v + 1e-5) * g_ref[...] + b_ref[...]
    w = (zn[:, 0:1] * mw_ref[0:1, :]
         + zn[:, 1:2] * mw_ref[1:2, :]
         + zn[:, 2:3] * mw_ref[2:3, :]
         + mb_ref[...])
    w_ref[...] = w
    sq_ref[...] = jnp.sum(w * w, axis=1, keepdims=True)


def _prep(nif, bn3_g, bn3_b, mlp_w, mlp_b):
    return pl.pallas_call(
        _prep_body,
        out_shape=(
            jax.ShapeDtypeStruct((N, 8), jnp.float32),
            jax.ShapeDtypeStruct((N, 1), jnp.float32),
        ),
    )(nif, bn3_g.reshape(1, 3), bn3_b.reshape(1, 3), mlp_w, mlp_b.reshape(1, 8))


def _build_a_body(s2_ref, w_ref, wt_ref, sq_ref, sqt_ref, a_ref, dis_ref):
    j = pl.program_id(1)
    nj = pl.num_programs(1)
    g = jnp.dot(w_ref[...], wt_ref[...], preferred_element_type=jnp.float32)
    d2 = jnp.maximum(sq_ref[...] + sqt_ref[...] - 2.0 * g, 0.0)
    wm = (jnp.exp(d2 * (-1.0 / 16.0)) + 1.0) * 0.5
    a = wm * s2_ref[...]
    a_ref[...] = a
    part = jnp.sum(a, axis=1, keepdims=True)

    @pl.when(j == 0)
    def _():
        dis_ref[...] = part

    @pl.when(j != 0)
    def _():
        dis_ref[...] += part

    @pl.when(j == nj - 1)
    def _():
        deg = dis_ref[...]
        dis_ref[...] = jnp.where(deg > 0.0, 1.0 / jnp.sqrt(deg), 0.0)


def _build_a(s2, w, wt, sq, sqt):
    return pl.pallas_call(
        _build_a_body,
        grid=(N // TM, N // TN),
        in_specs=[
            pl.BlockSpec((TM, TN), lambda m, j: (m, j)),
            pl.BlockSpec((TM, 8), lambda m, j: (m, 0)),
            pl.BlockSpec((8, TN), lambda m, j: (0, j)),
            pl.BlockSpec((TM, 1), lambda m, j: (m, 0)),
            pl.BlockSpec((1, TN), lambda m, j: (0, j)),
        ],
        out_specs=(
            pl.BlockSpec((TM, TN), lambda m, j: (m, j)),
            pl.BlockSpec((TM, 1), lambda m, j: (m, 0)),
        ),
        out_shape=(
            jax.ShapeDtypeStruct((N, N), jnp.float32),
            jax.ShapeDtypeStruct((N, 1), jnp.float32),
        ),
        compiler_params=pltpu.CompilerParams(
            dimension_semantics=("parallel", "arbitrary")),
    )(s2, w, wt, sq, sqt)


def _build_a0_body(w_ref, wt_ref, sq_ref, sqt_ref, h_ref, ht_ref, sqh_ref,
                   sqht_ref, a_ref, dis_ref):
    j = pl.program_id(1)
    nj = pl.num_programs(1)
    g8 = jnp.dot(w_ref[...], wt_ref[...], preferred_element_type=jnp.float32)
    d2w = jnp.maximum(sq_ref[...] + sqt_ref[...] - 2.0 * g8, 0.0)
    wm = (jnp.exp(d2w * (-1.0 / 16.0)) + 1.0) * 0.5
    gh = jnp.dot(h_ref[...], ht_ref[...], preferred_element_type=jnp.float32)
    d2h = jnp.maximum(sqh_ref[...] + sqht_ref[...] - 2.0 * gh, 0.0)
    s2n = jnp.exp(d2h * (-1.0 / 256.0))
    a = wm * s2n
    a_ref[...] = a
    part = jnp.sum(a, axis=1, keepdims=True)

    @pl.when(j == 0)
    def _():
        dis_ref[...] = part

    @pl.when(j != 0)
    def _():
        dis_ref[...] += part

    @pl.when(j == nj - 1)
    def _():
        deg = dis_ref[...]
        dis_ref[...] = jnp.where(deg > 0.0, 1.0 / jnp.sqrt(deg), 0.0)


def _build_a0(w, wt, sq, sqt, h, ht, sqh, sqht):
    return pl.pallas_call(
        _build_a0_body,
        grid=(N // TM, N // TN),
        in_specs=[
            pl.BlockSpec((TM, 8), lambda m, j: (m, 0)),
            pl.BlockSpec((8, TN), lambda m, j: (0, j)),
            pl.BlockSpec((TM, 1), lambda m, j: (m, 0)),
            pl.BlockSpec((1, TN), lambda m, j: (0, j)),
            pl.BlockSpec((TM, F), lambda m, j: (m, 0)),
            pl.BlockSpec((F, TN), lambda m, j: (0, j)),
            pl.BlockSpec((TM, 1), lambda m, j: (m, 0)),
            pl.BlockSpec((1, TN), lambda m, j: (0, j)),
        ],
        out_specs=(
            pl.BlockSpec((TM, TN), lambda m, j: (m, j)),
            pl.BlockSpec((TM, 1), lambda m, j: (m, 0)),
        ),
        out_shape=(
            jax.ShapeDtypeStruct((N, N), jnp.float32),
            jax.ShapeDtypeStruct((N, 1), jnp.float32),
        ),
        compiler_params=pltpu.CompilerParams(
            dimension_semantics=("parallel", "arbitrary")),
    )(w, wt, sq, sqt, h, ht, sqh, sqht)


def _prop_body(a_ref, t_ref, disk_ref, dism_ref, out_ref):
    k = pl.program_id(1)
    nk = pl.num_programs(1)
    part = jnp.dot(a_ref[...], disk_ref[...] * t_ref[...],
                   preferred_element_type=jnp.float32)

    @pl.when(k == 0)
    def _():
        out_ref[...] = part

    @pl.when(k != 0)
    def _():
        out_ref[...] += part

    @pl.when(k == nk - 1)
    def _():
        out_ref[...] = -dism_ref[...] * out_ref[...]


def _prop(a, t, dis):
    return pl.pallas_call(
        _prop_body,
        grid=(N // TM, N // TK),
        in_specs=[
            pl.BlockSpec((TM, TK), lambda m, k: (m, k)),
            pl.BlockSpec((TK, F), lambda m, k: (k, 0)),
            pl.BlockSpec((TK, 1), lambda m, k: (k, 0)),
            pl.BlockSpec((TM, 1), lambda m, k: (m, 0)),
        ],
        out_specs=pl.BlockSpec((TM, F), lambda m, k: (m, 0)),
        out_shape=jax.ShapeDtypeStruct((N, F), jnp.float32),
        compiler_params=pltpu.CompilerParams(
            dimension_semantics=("parallel", "arbitrary")),
    )(a, t, dis, dis)


def _prop_combine_body(a_ref, t_ref, disk_ref, dism_ref, x0_ref, tx1_ref,
                       w0_ref, w1_ref, w2_ref, b_ref, out_ref):
    k = pl.program_id(1)
    nk = pl.num_programs(1)
    part = jnp.dot(a_ref[...], disk_ref[...] * t_ref[...],
                   preferred_element_type=jnp.float32)

    @pl.when(k == 0)
    def _():
        out_ref[...] = part

    @pl.when(k != 0)
    def _():
        out_ref[...] += part

    @pl.when(k == nk - 1)
    def _():
        x0 = x0_ref[...]
        tx2 = -2.0 * dism_ref[...] * out_ref[...] - x0
        out_ref[...] = (
            jnp.dot(x0, w0_ref[...], preferred_element_type=jnp.float32)
            + jnp.dot(tx1_ref[...], w1_ref[...],
                      preferred_element_type=jnp.float32)
            + jnp.dot(tx2, w2_ref[...], preferred_element_type=jnp.float32)
            + b_ref[...])


def _prop_combine(a, tx1, dis, x0, w0, w1, w2, b):
    return pl.pallas_call(
        _prop_combine_body,
        grid=(N // TM, N // TK),
        in_specs=[
            pl.BlockSpec((TM, TK), lambda m, k: (m, k)),
            pl.BlockSpec((TK, F), lambda m, k: (k, 0)),
            pl.BlockSpec((TK, 1), lambda m, k: (k, 0)),
            pl.BlockSpec((TM, 1), lambda m, k: (m, 0)),
            pl.BlockSpec((TM, F), lambda m, k: (m, 0)),
            pl.BlockSpec((TM, F), lambda m, k: (m, 0)),
            pl.BlockSpec((F, F), lambda m, k: (0, 0)),
            pl.BlockSpec((F, F), lambda m, k: (0, 0)),
            pl.BlockSpec((F, F), lambda m, k: (0, 0)),
            pl.BlockSpec((1, F), lambda m, k: (0, 0)),
        ],
        out_specs=pl.BlockSpec((TM, F), lambda m, k: (m, 0)),
        out_shape=jax.ShapeDtypeStruct((N, F), jnp.float32),
        compiler_params=pltpu.CompilerParams(
            dimension_semantics=("parallel", "arbitrary")),
    )(a, tx1, dis, dis, x0, tx1, w0, w1, w2, b.reshape(1, F))


def _bn_act_body(x_ref, g_ref, b_ref, y_ref, sq_ref):
    x = x_ref[...]
    m = jnp.mean(x, axis=0, keepdims=True)
    v = jnp.mean((x - m) * (x - m), axis=0, keepdims=True)
    y = (x - m) / jnp.sqrt(v + 1e-5) * g_ref[...] + b_ref[...]
    y = jnp.where(y >= 0.0, y, 0.01 * y)
    y_ref[...] = y
    sq_ref[...] = jnp.sum(y * y, axis=1, keepdims=True)


def _bn_act(x, g, b):
    return pl.pallas_call(
        _bn_act_body,
        out_shape=(
            jax.ShapeDtypeStruct((N, F), jnp.float32),
            jax.ShapeDtypeStruct((N, 1), jnp.float32),
        ),
    )(x, g.reshape(1, F), b.reshape(1, F))


def _head_body(h_ref, w1_ref, b1_ref, g_ref, b_ref, w2_ref, b2_ref, out_ref):
    p = jnp.maximum(
        jnp.dot(h_ref[...], w1_ref[...], preferred_element_type=jnp.float32)
        + b1_ref[...], 0.0)
    m = jnp.mean(p, axis=0, keepdims=True)
    v = jnp.mean((p - m) * (p - m), axis=0, keepdims=True)
    p = (p - m) / jnp.sqrt(v + 1e-5) * g_ref[...] + b_ref[...]
    o = jnp.dot(p, w2_ref[...], preferred_element_type=jnp.float32) + b2_ref[...]
    out_ref[...] = jnp.maximum(o, 0.0)


def _head(h, p1_w, p1_b, bnp_g, bnp_b, p2_w, p2_b):
    return pl.pallas_call(
        _head_body,
        out_shape=jax.ShapeDtypeStruct((N, 16), jnp.float32),
    )(h, p1_w, p1_b.reshape(1, -1), bnp_g.reshape(1, -1),
      bnp_b.reshape(1, -1), p2_w, p2_b.reshape(1, -1))


def kernel(x, S2, no_image_feature, bn3_g, bn3_b, mlp_w, mlp_b, c1_w0, c1_w1,
           c1_w2, c1_b, c2_w0, c2_w1, c2_w2, c2_b, bn1_g, bn1_b, bn2_g, bn2_b,
           p1_w, p1_b, bnp_g, bnp_b, p2_w, p2_b):
    w, sq = _prep(no_image_feature, bn3_g, bn3_b, mlp_w, mlp_b)
    wt = w.T
    sqt = sq.T
    A, dis = _build_a(S2, w, wt, sq, sqt)

    def two_cheb(adj, d):
        tx1 = _prop(adj, x, d)
        h1p = _prop_combine(adj, tx1, d, x, c1_w0, c1_w1, c1_w2, c1_b)
        h1, _ = _bn_act(h1p, bn1_g, bn1_b)
        tx1b = _prop(adj, h1, d)
        h2p = _prop_combine(adj, tx1b, d, h1, c2_w0, c2_w1, c2_w2, c2_b)
        return _bn_act(h2p, bn2_g, bn2_b)

    h2, sqh = two_cheb(A, dis)
    A0, dis0 = _build_a0(w, wt, sq, sqt, h2, h2.T, sqh, sqh.T)
    h02, _ = two_cheb(A0, dis0)
    return _head(h02, p1_w, p1_b, bnp_g, bnp_b, p2_w, p2_b)
